# Initial kernel scaffold; baseline (speedup 1.0000x reference)
import jax
import jax.numpy as jnp
from jax.experimental import pallas as pl


def kernel(nodes, loc, edges, vel, edge_attr, charges, params):
    raise NotImplementedError("write your pallas kernel here")


# speedup vs baseline: 7.3358x; 7.3358x over previous
import functools

import jax
import jax.numpy as jnp
from jax import lax
from jax.experimental import pallas as pl
from jax.experimental.pallas import tpu as pltpu
from jax.experimental.pallas import tpu_sc as plsc

B = 2048
NPART = 5
N = B * NPART
E = B * 20
HID = 512
SLOPE = 0.2
EPS = 1e-6

R = 3 * N
E3 = 3 * E
NTILES = 32
NBATCH = E3 // (NTILES * 128)
ROWS_PER_TILE = R // 16
ZCHUNK = 480


def _make_seg_sum(F, nchunks):
    mesh = plsc.VectorSubcoreMesh(core_axis_name="c", subcore_axis_name="s",
                                  num_cores=2, num_subcores=16)

    @functools.partial(
        pl.kernel,
        out_type=jax.ShapeDtypeStruct((2, R, F * nchunks), jnp.float32),
        mesh=mesh,
        compiler_params=pltpu.CompilerParams(use_tc_tiling_on_sc=False),
        scratch_types=[
            pltpu.VMEM((NBATCH, 128), jnp.int32),
            pltpu.VMEM((NBATCH, 128), jnp.int32),
            pltpu.VMEM((128, F), jnp.float32),
            pltpu.VMEM((ZCHUNK, F), jnp.float32),
            pltpu.VMEM_SHARED((R, F), jnp.float32),
            pltpu.SemaphoreType.DMA,
        ],
    )
    def seg_sum(x_hbm, gidx_hbm, sidx_hbm, zeros_hbm, out_hbm,
                gidx_v, sidx_v, buf, zbuf, acc, sem):
        cid = lax.axis_index("c")
        sid = lax.axis_index("s")
        wid = sid * 2 + cid
        r0 = sid * ROWS_PER_TILE

        pltpu.sync_copy(zeros_hbm, zbuf)
        pltpu.sync_copy(sidx_hbm.at[wid], sidx_v)

        for f in range(nchunks):
            for j in range(ROWS_PER_TILE // ZCHUNK):
                pltpu.sync_copy(zbuf, acc.at[pl.ds(r0 + j * ZCHUNK, ZCHUNK)])
            pltpu.sync_copy(gidx_hbm.at[f, wid], gidx_v)
            plsc.subcore_barrier()

            @pl.loop(0, NBATCH)
            def _(b):
                pltpu.async_copy(x_hbm.at[gidx_v.at[b]], buf, sem).wait()
                pltpu.sync_copy(buf, acc.at[sidx_v.at[b]], add=True)

            plsc.subcore_barrier()
            pltpu.sync_copy(
                acc.at[pl.ds(r0, ROWS_PER_TILE)],
                out_hbm.at[cid, pl.ds(r0, ROWS_PER_TILE), pl.ds(f * F, F)])
            plsc.subcore_barrier()

    return seg_sum


@functools.cache
def _seg_sum(F, nchunks):
    return _make_seg_sum(F, nchunks)


def _seg_sum_16(*args):
    return _seg_sum(16, 1)(*args)


def _seg_sum_512(*args):
    return _seg_sum(32, 16)(*args)


_BN = 256


def _feats_body(loc_ref, vel_ref, ch_ref, p_ref, out_ref):
    l = loc_ref[...]
    v = vel_ref[...]
    c = ch_ref[...]
    m = jnp.dot(p_ref[...], l, preferred_element_type=jnp.float32)
    cl = l - m
    a0 = cl[:, 1:2] * v[:, 2:3] - cl[:, 2:3] * v[:, 1:2]
    a1 = cl[:, 2:3] * v[:, 0:1] - cl[:, 0:1] * v[:, 2:3]
    a2 = cl[:, 0:1] * v[:, 1:2] - cl[:, 1:2] * v[:, 0:1]
    ang = jnp.concatenate([a0, a1, a2], axis=1)
    clc = cl * c
    bn = l.shape[0]
    ones = jnp.ones((bn, 1), jnp.float32)
    pad = jnp.zeros((bn, 11), jnp.float32)
    planes = [
        jnp.concatenate([cl[:, k:k + 1], v[:, k:k + 1], ang[:, k:k + 1],
                         clc[:, k:k + 1], ones, pad], axis=1)
        for k in range(3)
    ]
    out_ref[...] = jnp.stack(planes, axis=0)


def _vn_tail(s, d, bn):
    s3 = s.reshape(3, bn, HID)
    d3 = d.reshape(3, bn, HID)
    dot = jnp.sum(s3 * d3, axis=0)
    dsq = jnp.sum(d3 * d3, axis=0)
    coef = jnp.where(dot >= 0, 0.0, dot / (dsq + EPS))
    return s3 - (1.0 - SLOPE) * coef[None] * d3


def _layer0_body(x_ref, p_ref, wi_ref, wp_ref, wd_ref, bs_ref,
                 out_ref, inv_ref):
    bn = x_ref.shape[1]
    xb = x_ref[...]
    pr = p_ref[...]
    ps = pr[0] + pr[1]
    cnt = ps[0:1, :, 4:5]
    inv = 1.0 / jnp.maximum(cnt, 1.0)
    pooled = ps[:, :, 0:4] * inv
    x2 = xb[:, :, 0:4].reshape(3 * bn, 4)
    p2 = pooled.reshape(3 * bn, 4)
    s = (jnp.dot(x2, wi_ref[...], preferred_element_type=jnp.float32)
         + jnp.dot(p2, wp_ref[...], preferred_element_type=jnp.float32)
         + bs_ref[...])
    d = jnp.dot(s, wd_ref[...], preferred_element_type=jnp.float32)
    out_ref[...] = _vn_tail(s, d, bn)
    inv_ref[...] = inv


def _layer_body(x_ref, p_ref, inv_ref, wi_ref, wp_ref, wd_ref, bs_ref,
                out_ref):
    bn = x_ref.shape[1]
    xb = x_ref[...]
    pr = p_ref[...]
    ps = pr[0] + pr[1]
    pooled = ps * inv_ref[...]
    x2 = xb.reshape(3 * bn, HID)
    p2 = pooled.reshape(3 * bn, HID)
    s = (jnp.dot(x2, wi_ref[...], preferred_element_type=jnp.float32)
         + jnp.dot(p2, wp_ref[...], preferred_element_type=jnp.float32)
         + bs_ref[...])
    d = jnp.dot(s, wd_ref[...], preferred_element_type=jnp.float32)
    out_ref[...] = _vn_tail(s, d, bn) + xb


def _layer3_body(x_ref, p_ref, inv_ref, wi_ref, wp_ref, wd_ref, bs_ref,
                 wo_ref, bo_ref, out_ref):
    bn = x_ref.shape[1]
    xb = x_ref[...]
    pr = p_ref[...]
    ps = pr[0] + pr[1]
    pooled = ps * inv_ref[...]
    x2 = xb.reshape(3 * bn, HID)
    p2 = pooled.reshape(3 * bn, HID)
    s = (jnp.dot(x2, wi_ref[...], preferred_element_type=jnp.float32)
         + jnp.dot(p2, wp_ref[...], preferred_element_type=jnp.float32)
         + bs_ref[...])
    d = jnp.dot(s, wd_ref[...], preferred_element_type=jnp.float32)
    y = _vn_tail(s, d, bn) + xb
    out_ref[...] = jnp.sum(y * wo_ref[...][None], axis=2) + bo_ref[0, 0]


def _full(shape):
    return pl.BlockSpec(shape, lambda i: (0,) * len(shape))


_BNF = 1280


def _feats_call(loc, vel, charges, pmat):
    grid = (N // _BNF,)
    return pl.pallas_call(
        _feats_body,
        grid=grid,
        in_specs=[
            pl.BlockSpec((_BNF, 3), lambda i: (i, 0)),
            pl.BlockSpec((_BNF, 3), lambda i: (i, 0)),
            pl.BlockSpec((_BNF, 1), lambda i: (i, 0)),
            _full((_BNF, _BNF)),
        ],
        out_specs=pl.BlockSpec((3, _BNF, 16), lambda i: (0, i, 0)),
        out_shape=jax.ShapeDtypeStruct((3, N, 16), jnp.float32),
    )(loc, vel, charges, pmat)


def _layer0_call(x16, p16, wi, wp, wd, bs):
    grid = (N // _BN,)
    return pl.pallas_call(
        _layer0_body,
        grid=grid,
        in_specs=[
            pl.BlockSpec((3, _BN, 16), lambda i: (0, i, 0)),
            pl.BlockSpec((2, 3, _BN, 16), lambda i: (0, 0, i, 0)),
            _full((4, HID)), _full((4, HID)), _full((HID, HID)),
            _full((1, HID)),
        ],
        out_specs=[
            pl.BlockSpec((3, _BN, HID), lambda i: (0, i, 0)),
            pl.BlockSpec((1, _BN, 1), lambda i: (0, i, 0)),
        ],
        out_shape=[
            jax.ShapeDtypeStruct((3, N, HID), jnp.float32),
            jax.ShapeDtypeStruct((1, N, 1), jnp.float32),
        ],
    )(x16, p16, wi, wp, wd, bs)


def _layer_call(x, p, inv, wi, wp, wd, bs):
    grid = (N // _BN,)
    return pl.pallas_call(
        _layer_body,
        grid=grid,
        in_specs=[
            pl.BlockSpec((3, _BN, HID), lambda i: (0, i, 0)),
            pl.BlockSpec((2, 3, _BN, HID), lambda i: (0, 0, i, 0)),
            pl.BlockSpec((1, _BN, 1), lambda i: (0, i, 0)),
            _full((HID, HID)), _full((HID, HID)), _full((HID, HID)),
            _full((1, HID)),
        ],
        out_specs=pl.BlockSpec((3, _BN, HID), lambda i: (0, i, 0)),
        out_shape=jax.ShapeDtypeStruct((3, N, HID), jnp.float32),
    )(x, p, inv, wi, wp, wd, bs)


def _layer3_call(x, p, inv, wi, wp, wd, bs, wo, bo):
    grid = (N // _BN,)
    return pl.pallas_call(
        _layer3_body,
        grid=grid,
        in_specs=[
            pl.BlockSpec((3, _BN, HID), lambda i: (0, i, 0)),
            pl.BlockSpec((2, 3, _BN, HID), lambda i: (0, 0, i, 0)),
            pl.BlockSpec((1, _BN, 1), lambda i: (0, i, 0)),
            _full((HID, HID)), _full((HID, HID)), _full((HID, HID)),
            _full((1, HID)), _full((1, HID)), _full((1, 1)),
        ],
        out_specs=pl.BlockSpec((3, _BN), lambda i: (0, i)),
        out_shape=jax.ShapeDtypeStruct((3, N), jnp.float32),
    )(x, p, inv, wi, wp, wd, bs, wo, bo)


def kernel(nodes, loc, edges, vel, edge_attr, charges, params):
    del nodes, edge_attr
    i32 = jnp.int32

    koff = (jnp.arange(3, dtype=i32) * N)[:, None]
    src3 = (edges[0][None, :].astype(i32) + koff).reshape(NTILES, NBATCH, 128)
    dst3 = (edges[1][None, :].astype(i32) + koff).reshape(NTILES, NBATCH, 128)
    gidx16 = src3[None]
    chunk = jnp.arange(16, dtype=i32)[:, None, None, None]
    gidx512 = src3[None] * 16 + chunk

    zeros16 = jnp.zeros((ZCHUNK, 16), jnp.float32)
    zeros64 = jnp.zeros((ZCHUNK, 32), jnp.float32)

    rows = jnp.arange(_BNF) // NPART
    pmat = (rows[:, None] == rows[None, :]).astype(jnp.float32) / NPART

    p = params
    wiT = [p["Wi%d" % i].T for i in range(4)]
    wpT = [p["Wp%d" % i].T for i in range(4)]
    wdT = [p["Wd%d" % i].T for i in range(4)]
    bs = [(p["bi%d" % i] + p["bp%d" % i]).reshape(1, HID) for i in range(4)]
    wo = p["Wo"].reshape(1, HID)
    bo = p["bo"].reshape(1, 1)

    feats16 = _feats_call(loc, vel, charges, pmat)
    p16 = _seg_sum_16(feats16.reshape(R, 16), gidx16, dst3, zeros16)
    p16 = p16.reshape(2, 3, N, 16)
    x, inv = _layer0_call(feats16, p16, wiT[0], wpT[0], wdT[0], bs[0])

    for i in (1, 2):
        ps = _seg_sum_512(x.reshape(R * 16, 32), gidx512, dst3, zeros64)
        ps = ps.reshape(2, 3, N, HID)
        x = _layer_call(x, ps, inv, wiT[i], wpT[i], wdT[i], bs[i])

    ps = _seg_sum_512(x.reshape(R * 16, 32), gidx512, dst3, zeros64)
    ps = ps.reshape(2, 3, N, HID)
    out3 = _layer3_call(x, ps, inv, wiT[3], wpT[3], wdT[3], bs[3], wo, bo)
    return out3.T
